# Initial kernel scaffold; baseline (speedup 1.0000x reference)
import jax
import jax.numpy as jnp
from jax.experimental import pallas as pl


def kernel(drug_x, protein_x, edge_index, rev_edge_index, W_drug_lin, b_drug_lin, W_protein_lin, b_protein_lin, conv_W_dp, conv_b_dp, conv_W_pd, conv_b_pd, W_link, b_link):
    raise NotImplementedError("write your pallas kernel here")


# speedup vs baseline: 33.0596x; 33.0596x over previous
import functools

import jax
import jax.numpy as jnp
from jax import lax
from jax.experimental import pallas as pl
from jax.experimental.pallas import tpu as pltpu
from jax.experimental.pallas import tpu_sc as plsc

NACC = 10240
LCH = 128
NT = 32


def _make_deg_kernel(ch):
    mesh = plsc.VectorSubcoreMesh(core_axis_name="c", subcore_axis_name="s")

    @functools.partial(
        pl.kernel, mesh=mesh,
        out_type=jax.ShapeDtypeStruct((2, 4, NACC), jnp.float32),
        scratch_types=[
            pltpu.VMEM((ch, LCH), jnp.int32),
            pltpu.VMEM((LCH,), jnp.float32),
            pltpu.VMEM((NACC,), jnp.float32),
            pltpu.VMEM_SHARED((NACC,), jnp.float32),
            pltpu.VMEM_SHARED((NACC,), jnp.float32),
            pltpu.VMEM_SHARED((NACC,), jnp.float32),
            pltpu.VMEM_SHARED((NACC,), jnp.float32),
        ],
    )
    def deg_kernel(idx_hbm, out_hbm, idx_v, ones_v, zro_v, a0, a1, a2, a3):
        cid = lax.axis_index("c")
        sid = lax.axis_index("s")
        wid = sid * 2 + cid
        accs = [a0, a1, a2, a3]
        for i in range(LCH // 16):
            ones_v[pl.ds(i * 16, 16)] = jnp.ones((16,), jnp.float32)

        @pl.when(sid == 0)
        def _():
            def zb(i, c):
                zro_v[pl.ds(i * 16, 16)] = jnp.zeros((16,), jnp.float32)
                return c
            lax.fori_loop(0, NACC // 16, zb, 0)
            for a in range(4):
                pltpu.sync_copy(zro_v, accs[a])

        plsc.subcore_barrier()
        for a in range(4):
            pltpu.sync_copy(idx_hbm.at[a, wid], idx_v)

            def sb(j, c, _acc=accs[a]):
                pltpu.sync_copy(ones_v, _acc.at[idx_v.at[j]], add=True)
                return c
            lax.fori_loop(0, ch, sb, 0)
        plsc.subcore_barrier()

        @pl.when(sid == 0)
        def _():
            for a in range(4):
                pltpu.sync_copy(accs[a], out_hbm.at[cid, a])

    return deg_kernel


def _make_edge_kernel(ch):
    mesh = plsc.VectorSubcoreMesh(core_axis_name="c", subcore_axis_name="s")

    @functools.partial(
        pl.kernel, mesh=mesh,
        out_type=jax.ShapeDtypeStruct((2, 2, NACC), jnp.float32),
        scratch_types=[
            pltpu.VMEM((ch, LCH), jnp.int32),
            pltpu.VMEM((ch, LCH), jnp.int32),
            pltpu.VMEM((LCH,), jnp.float32),
            pltpu.VMEM((NACC,), jnp.float32),
            pltpu.VMEM_SHARED((NACC,), jnp.float32),
            pltpu.VMEM_SHARED((NACC,), jnp.float32),
            pltpu.SemaphoreType.DMA,
        ],
    )
    def edge_kernel(q0_hbm, q1_hbm, idx_hbm, out_hbm, sidx_v, didx_v, val_v,
                    zro_v, acc0, acc1, sem):
        cid = lax.axis_index("c")
        sid = lax.axis_index("s")
        wid = sid * 2 + cid
        accs = [acc0, acc1]
        qs = [q0_hbm, q1_hbm]

        @pl.when(sid == 0)
        def _():
            def zb(i, c):
                zro_v[pl.ds(i * 16, 16)] = jnp.zeros((16,), jnp.float32)
                return c
            lax.fori_loop(0, NACC // 16, zb, 0)
            for s in range(2):
                pltpu.sync_copy(zro_v, accs[s])

        plsc.subcore_barrier()
        for s in range(2):
            pltpu.sync_copy(idx_hbm.at[2 * s, wid], sidx_v)
            pltpu.sync_copy(idx_hbm.at[2 * s + 1, wid], didx_v)

            def eb(j, c, _acc=accs[s], _q=qs[s]):
                pltpu.async_copy(_q.at[sidx_v.at[j]], val_v, sem).wait()
                pltpu.sync_copy(val_v, _acc.at[didx_v.at[j]], add=True)
                return c
            lax.fori_loop(0, ch, eb, 0)
        plsc.subcore_barrier()

        @pl.when(sid == 0)
        def _():
            for s in range(2):
                pltpu.sync_copy(accs[s], out_hbm.at[cid, s])

    return edge_kernel


def _q_kernel(dx_ref, px_ref, wdp_ref, w1_ref, wpd_ref, w2_ref, deg_ref, q_ref):
    u1 = jnp.dot(wdp_ref[...], w1_ref[...], preferred_element_type=jnp.float32)
    u2 = jnp.dot(wpd_ref[...], w2_ref[...], preferred_element_type=jnp.float32)
    s_d = jnp.dot(dx_ref[...], u1, preferred_element_type=jnp.float32)[:, 0]
    s_p = jnp.dot(px_ref[...], u2, preferred_element_type=jnp.float32)[:, 0]
    deg = deg_ref[0] + deg_ref[1]
    q_ref[0, :] = s_d * lax.rsqrt(jnp.maximum(deg[0], 1.0))
    q_ref[1, :] = s_p * lax.rsqrt(jnp.maximum(deg[2], 1.0))


def _fin_kernel(t_ref, deg_ref, bdp_ref, bpd_ref, w1_ref, w2_ref, bl_ref, o_ref):
    c1 = (jnp.sum(bdp_ref[...] * w1_ref[...])
          + jnp.sum(bpd_ref[...] * w2_ref[...]) + bl_ref[0, 0])
    t0 = t_ref[0, 0] + t_ref[1, 0]
    t1 = t_ref[0, 1] + t_ref[1, 1]
    deg = deg_ref[0] + deg_ref[1]
    r1 = lax.rsqrt(jnp.maximum(deg[1], 1.0))
    r2 = lax.rsqrt(jnp.maximum(deg[3], 1.0))
    z = t0 * r1 + t1 * r2 + c1
    o_ref[...] = 1.0 / (1.0 + jnp.exp(-z))


def kernel(drug_x, protein_x, edge_index, rev_edge_index, W_drug_lin,
           b_drug_lin, W_protein_lin, b_protein_lin, conv_W_dp, conv_b_dp,
           conv_W_pd, conv_b_pd, W_link, b_link):
    n = drug_x.shape[0]
    d_h = conv_W_dp.shape[2]
    e = edge_index.shape[1]
    ch = -(-e // (NT * LCH))
    epad = NT * ch * LCH

    w1 = W_link[:d_h]
    w2 = W_link[d_h:]
    wdp = conv_W_dp[-1]
    wpd = conv_W_pd[-1]

    def prep(v):
        pad = jnp.full((epad - e,), n, jnp.int32)
        return jnp.concatenate([v.astype(jnp.int32), pad]).reshape(NT, ch, LCH)

    idx_all = jnp.stack([prep(edge_index[0]), prep(edge_index[1]),
                         prep(rev_edge_index[0]), prep(rev_edge_index[1])])

    deg_part = _make_deg_kernel(ch)(idx_all)

    blk = 1024
    nb = NACC // blk
    q = pl.pallas_call(
        _q_kernel,
        grid=(nb,),
        in_specs=[
            pl.BlockSpec((blk, drug_x.shape[1]), lambda i: (i, 0)),
            pl.BlockSpec((blk, protein_x.shape[1]), lambda i: (i, 0)),
            pl.BlockSpec(wdp.shape, lambda i: (0, 0)),
            pl.BlockSpec(w1.shape, lambda i: (0, 0)),
            pl.BlockSpec(wpd.shape, lambda i: (0, 0)),
            pl.BlockSpec(w2.shape, lambda i: (0, 0)),
            pl.BlockSpec((2, 4, blk), lambda i: (0, 0, i)),
        ],
        out_specs=pl.BlockSpec((2, blk), lambda i: (0, i)),
        out_shape=jax.ShapeDtypeStruct((2, NACC), jnp.float32),
    )(drug_x, protein_x, wdp, w1, wpd, w2, deg_part)

    t_part = _make_edge_kernel(ch)(q[0], q[1], idx_all)

    out_full = pl.pallas_call(
        _fin_kernel,
        out_shape=jax.ShapeDtypeStruct((NACC,), jnp.float32),
    )(t_part, deg_part,
      conv_b_dp[-1].reshape(2, d_h // 2), conv_b_pd[-1].reshape(2, d_h // 2),
      w1.reshape(2, d_h // 2), w2.reshape(2, d_h // 2),
      b_link.reshape(1, 1))

    return out_full[:n].reshape(n, 1)


# speedup vs baseline: 40.9870x; 1.2398x over previous
import functools

import jax
import jax.numpy as jnp
from jax import lax
from jax.experimental import pallas as pl
from jax.experimental.pallas import tpu as pltpu
from jax.experimental.pallas import tpu_sc as plsc

NACC = 10240
LCH = 128
NT = 32
SEG = NACC // 16


def _zero_fill(buf):
    for i in range(buf.shape[0] // 16):
        buf[pl.ds(i * 16, 16)] = jnp.zeros((16,), jnp.float32)


def _make_deg_kernel(ch):
    mesh = plsc.VectorSubcoreMesh(core_axis_name="c", subcore_axis_name="s")

    @functools.partial(
        pl.kernel, mesh=mesh,
        out_type=jax.ShapeDtypeStruct((2, 4, NACC), jnp.float32),
        scratch_types=[
            pltpu.VMEM((4, ch, LCH), jnp.int32),
            pltpu.VMEM((LCH,), jnp.float32),
            pltpu.VMEM((SEG,), jnp.float32),
            pltpu.VMEM_SHARED((NACC,), jnp.float32),
            pltpu.VMEM_SHARED((NACC,), jnp.float32),
            pltpu.VMEM_SHARED((NACC,), jnp.float32),
            pltpu.VMEM_SHARED((NACC,), jnp.float32),
            pltpu.SemaphoreType.DMA,
        ],
    )
    def deg_kernel(idx_hbm, out_hbm, idx_v, ones_v, zseg_v, a0, a1, a2, a3, sem):
        cid = lax.axis_index("c")
        sid = lax.axis_index("s")
        wid = sid * 2 + cid
        accs = [a0, a1, a2, a3]
        for i in range(LCH // 16):
            ones_v[pl.ds(i * 16, 16)] = jnp.ones((16,), jnp.float32)
        _zero_fill(zseg_v)
        for a in range(4):
            pltpu.sync_copy(idx_hbm.at[a, wid], idx_v.at[a])
            pltpu.sync_copy(zseg_v, accs[a].at[pl.ds(sid * SEG, SEG)])
        plsc.subcore_barrier()
        for a in range(4):
            def fire(j, c, _acc=accs[a], _a=a):
                pltpu.async_copy(ones_v, _acc.at[idx_v.at[_a, j]], sem, add=True)
                return c
            lax.fori_loop(0, ch, fire, 0)
        for a in range(4):
            def drain(j, c, _acc=accs[a], _a=a):
                pltpu.make_async_copy(ones_v, _acc.at[idx_v.at[_a, j]], sem).wait()
                return c
            lax.fori_loop(0, ch, drain, 0)
        plsc.subcore_barrier()

        @pl.when(sid == 0)
        def _():
            for a in range(4):
                pltpu.sync_copy(accs[a], out_hbm.at[cid, a])

    return deg_kernel


def _make_edge_kernel(ch):
    mesh = plsc.VectorSubcoreMesh(core_axis_name="c", subcore_axis_name="s")

    @functools.partial(
        pl.kernel, mesh=mesh,
        out_type=jax.ShapeDtypeStruct((2, 2, NACC), jnp.float32),
        scratch_types=[
            pltpu.VMEM((2, ch, LCH), jnp.int32),
            pltpu.VMEM((2, ch, LCH), jnp.int32),
            pltpu.VMEM((2, ch, LCH), jnp.float32),
            pltpu.VMEM((SEG,), jnp.float32),
            pltpu.VMEM_SHARED((NACC,), jnp.float32),
            pltpu.VMEM_SHARED((NACC,), jnp.float32),
            pltpu.SemaphoreType.DMA,
            pltpu.SemaphoreType.DMA,
        ],
    )
    def edge_kernel(q0_hbm, q1_hbm, idx_hbm, out_hbm, sidx_v, didx_v, val_v,
                    zseg_v, acc0, acc1, gsem, ssem):
        cid = lax.axis_index("c")
        sid = lax.axis_index("s")
        wid = sid * 2 + cid
        accs = [acc0, acc1]
        qs = [q0_hbm, q1_hbm]
        _zero_fill(zseg_v)
        for s in range(2):
            pltpu.sync_copy(idx_hbm.at[2 * s, wid], sidx_v.at[s])
            pltpu.sync_copy(idx_hbm.at[2 * s + 1, wid], didx_v.at[s])
            pltpu.sync_copy(zseg_v, accs[s].at[pl.ds(sid * SEG, SEG)])
        plsc.subcore_barrier()
        for s in range(2):
            def gfire(j, c, _q=qs[s], _s=s):
                pltpu.async_copy(_q.at[sidx_v.at[_s, j]], val_v.at[_s, j], gsem)
                return c
            lax.fori_loop(0, ch, gfire, 0)
        for s in range(2):
            def gdrain(j, c, _q=qs[s], _s=s):
                pltpu.make_async_copy(_q.at[sidx_v.at[_s, j]], val_v.at[_s, j],
                                      gsem).wait()
                return c
            lax.fori_loop(0, ch, gdrain, 0)
        for s in range(2):
            def sfire(j, c, _acc=accs[s], _s=s):
                pltpu.async_copy(val_v.at[_s, j], _acc.at[didx_v.at[_s, j]],
                                 ssem, add=True)
                return c
            lax.fori_loop(0, ch, sfire, 0)
        for s in range(2):
            def sdrain(j, c, _acc=accs[s], _s=s):
                pltpu.make_async_copy(val_v.at[_s, j], _acc.at[didx_v.at[_s, j]],
                                      ssem).wait()
                return c
            lax.fori_loop(0, ch, sdrain, 0)
        plsc.subcore_barrier()

        @pl.when(sid == 0)
        def _():
            for s in range(2):
                pltpu.sync_copy(accs[s], out_hbm.at[cid, s])

    return edge_kernel


def _q_kernel(dx_ref, px_ref, wdp_ref, w1_ref, wpd_ref, w2_ref, deg_ref, q_ref):
    u1 = jnp.dot(wdp_ref[...], w1_ref[...], preferred_element_type=jnp.float32)
    u2 = jnp.dot(wpd_ref[...], w2_ref[...], preferred_element_type=jnp.float32)
    s_d = jnp.dot(dx_ref[...], u1, preferred_element_type=jnp.float32)[:, 0]
    s_p = jnp.dot(px_ref[...], u2, preferred_element_type=jnp.float32)[:, 0]
    deg = deg_ref[0] + deg_ref[1]
    q_ref[0, :] = s_d * lax.rsqrt(jnp.maximum(deg[0], 1.0))
    q_ref[1, :] = s_p * lax.rsqrt(jnp.maximum(deg[2], 1.0))


def _fin_kernel(t_ref, deg_ref, bdp_ref, bpd_ref, w1_ref, w2_ref, bl_ref, o_ref):
    c1 = (jnp.sum(bdp_ref[...] * w1_ref[...])
          + jnp.sum(bpd_ref[...] * w2_ref[...]) + bl_ref[0, 0])
    t0 = t_ref[0, 0] + t_ref[1, 0]
    t1 = t_ref[0, 1] + t_ref[1, 1]
    deg = deg_ref[0] + deg_ref[1]
    r1 = lax.rsqrt(jnp.maximum(deg[1], 1.0))
    r2 = lax.rsqrt(jnp.maximum(deg[3], 1.0))
    z = t0 * r1 + t1 * r2 + c1
    o_ref[...] = 1.0 / (1.0 + jnp.exp(-z))


def kernel(drug_x, protein_x, edge_index, rev_edge_index, W_drug_lin,
           b_drug_lin, W_protein_lin, b_protein_lin, conv_W_dp, conv_b_dp,
           conv_W_pd, conv_b_pd, W_link, b_link):
    n = drug_x.shape[0]
    d_h = conv_W_dp.shape[2]
    e = edge_index.shape[1]
    ch = -(-e // (NT * LCH))
    epad = NT * ch * LCH

    w1 = W_link[:d_h]
    w2 = W_link[d_h:]
    wdp = conv_W_dp[-1]
    wpd = conv_W_pd[-1]

    def prep(v):
        pad = jnp.full((epad - e,), n, jnp.int32)
        return jnp.concatenate([v.astype(jnp.int32), pad]).reshape(NT, ch, LCH)

    idx_all = jnp.stack([prep(edge_index[0]), prep(edge_index[1]),
                         prep(rev_edge_index[0]), prep(rev_edge_index[1])])

    deg_part = _make_deg_kernel(ch)(idx_all)

    blk = 1024
    nb = NACC // blk
    q = pl.pallas_call(
        _q_kernel,
        grid=(nb,),
        in_specs=[
            pl.BlockSpec((blk, drug_x.shape[1]), lambda i: (i, 0)),
            pl.BlockSpec((blk, protein_x.shape[1]), lambda i: (i, 0)),
            pl.BlockSpec(wdp.shape, lambda i: (0, 0)),
            pl.BlockSpec(w1.shape, lambda i: (0, 0)),
            pl.BlockSpec(wpd.shape, lambda i: (0, 0)),
            pl.BlockSpec(w2.shape, lambda i: (0, 0)),
            pl.BlockSpec((2, 4, blk), lambda i: (0, 0, i)),
        ],
        out_specs=pl.BlockSpec((2, blk), lambda i: (0, i)),
        out_shape=jax.ShapeDtypeStruct((2, NACC), jnp.float32),
    )(drug_x, protein_x, wdp, w1, wpd, w2, deg_part)

    t_part = _make_edge_kernel(ch)(q[0], q[1], idx_all)

    out_full = pl.pallas_call(
        _fin_kernel,
        out_shape=jax.ShapeDtypeStruct((NACC,), jnp.float32),
    )(t_part, deg_part,
      conv_b_dp[-1].reshape(2, d_h // 2), conv_b_pd[-1].reshape(2, d_h // 2),
      w1.reshape(2, d_h // 2), w2.reshape(2, d_h // 2),
      b_link.reshape(1, 1))

    return out_full[:n].reshape(n, 1)
